# Initial kernel scaffold; baseline (speedup 1.0000x reference)
import jax
import jax.numpy as jnp
from jax.experimental import pallas as pl


def kernel(moe_inp, gate_w, gate_b, w1, b1, w2, b2):
    raise NotImplementedError("write your pallas kernel here")


# speedup vs baseline: 1.4783x; 1.4783x over previous
import functools

import jax
import jax.numpy as jnp
from jax import lax
from jax.experimental import pallas as pl
from jax.experimental.pallas import tpu as pltpu

NUM_E = 16
D = 1024
F = 2048
K = 2
T = 2048
S = T * K
B = 128
NB_MAX = S // B + NUM_E
NPAD = NB_MAX * B


def _gate_body(x_ref, gw_ref, gb_ref, topi_ref, g_ref):
    logits = jnp.dot(x_ref[...], gw_ref[...],
                     preferred_element_type=jnp.float32) + gb_ref[0][None, :]
    idx16 = lax.broadcasted_iota(jnp.int32, logits.shape, 1)
    m1 = jnp.max(logits, axis=1, keepdims=True)
    i1 = jnp.min(jnp.where(logits == m1, idx16, 9999), axis=1, keepdims=True)
    masked = jnp.where(idx16 == i1, -1e30, logits)
    m2 = jnp.max(masked, axis=1, keepdims=True)
    i2 = jnp.min(jnp.where(masked == m2, idx16, 9999), axis=1, keepdims=True)
    e2 = jnp.exp(m2 - m1)
    g1 = 1.0 / (1.0 + e2)
    topi_ref[:, 0:1] = i1
    topi_ref[:, 1:2] = i2
    g_ref[:, 0:1] = g1
    g_ref[:, 1:2] = 1.0 - g1


def _gate(x, gate_w, gate_b):
    TB = 256
    return pl.pallas_call(
        _gate_body,
        grid=(T // TB,),
        in_specs=[
            pl.BlockSpec((TB, D), lambda i: (i, 0)),
            pl.BlockSpec((D, NUM_E), lambda i: (0, 0)),
            pl.BlockSpec((1, NUM_E), lambda i: (0, 0)),
        ],
        out_specs=[
            pl.BlockSpec((TB, K), lambda i: (i, 0)),
            pl.BlockSpec((TB, K), lambda i: (i, 0)),
        ],
        out_shape=[
            jax.ShapeDtypeStruct((T, K), jnp.int32),
            jax.ShapeDtypeStruct((T, K), jnp.float32),
        ],
    )(x, gate_w, gate_b.reshape(1, NUM_E))


def _routing(topi):
    e_flat = topi.reshape(-1)
    onehot = (e_flat[:, None] == jnp.arange(NUM_E)[None, :]).astype(jnp.int32)
    cnt_cum = jnp.cumsum(onehot, axis=0)
    counts = cnt_cum[-1]
    rank = jnp.sum(onehot * cnt_cum, axis=1) - 1
    nb_e = (counts + B - 1) // B
    cum_nb = jnp.cumsum(nb_e)
    nb = cum_nb[-1].astype(jnp.int32)
    padded_off = jnp.concatenate(
        [jnp.zeros(1, jnp.int32), cum_nb[:-1].astype(jnp.int32)]) * B
    dest_pos = padded_off[e_flat] + rank
    arange_s = jnp.arange(S, dtype=jnp.int32)
    row_gather = jnp.zeros(NPAD, jnp.int32).at[dest_pos].set(arange_s // K)
    be = jnp.searchsorted(cum_nb, jnp.minimum(jnp.arange(NB_MAX), nb - 1),
                          side="right").astype(jnp.int32)
    dest_slot = jnp.full(NPAD, S, jnp.int32).at[dest_pos].set(arange_s)
    return row_gather, be, dest_slot, nb


def _ffn_body(be_ref, xb_ref, nb_ref, xs_ref, w1_ref, b1_ref, w2_ref, b2_ref,
              out_ref):
    bidx = pl.program_id(0)

    @pl.when(bidx < nb_ref[0])
    def _():
        h = jnp.dot(xs_ref[...], w1_ref[0],
                    preferred_element_type=jnp.float32) + b1_ref[0, 0][None, :]
        h = jnp.maximum(h, 0.0)
        y = jnp.dot(h, w2_ref[0],
                    preferred_element_type=jnp.float32) + b2_ref[0, 0][None, :]
        out_ref[...] = y


def _ffn(xs, w1, b1, w2, b2, be, nb):
    xb = jnp.minimum(jnp.arange(NB_MAX, dtype=jnp.int32), nb - 1)
    grid_spec = pltpu.PrefetchScalarGridSpec(
        num_scalar_prefetch=3,
        grid=(NB_MAX,),
        in_specs=[
            pl.BlockSpec((B, D), lambda b, be, xb, nbv: (xb[b], 0)),
            pl.BlockSpec((1, D, F), lambda b, be, xb, nbv: (be[b], 0, 0)),
            pl.BlockSpec((1, 1, F), lambda b, be, xb, nbv: (be[b], 0, 0)),
            pl.BlockSpec((1, F, D), lambda b, be, xb, nbv: (be[b], 0, 0)),
            pl.BlockSpec((1, 1, D), lambda b, be, xb, nbv: (be[b], 0, 0)),
        ],
        out_specs=pl.BlockSpec((B, D), lambda b, be, xb, nbv: (xb[b], 0)),
    )
    return pl.pallas_call(
        _ffn_body,
        grid_spec=grid_spec,
        out_shape=jax.ShapeDtypeStruct((NPAD, D), jnp.float32),
        compiler_params=pltpu.CompilerParams(
            dimension_semantics=("arbitrary",)),
    )(be, xb, nb.reshape(1), xs, w1, b1.reshape(NUM_E, 1, F), w2,
      b2.reshape(NUM_E, 1, D))


def _combine_body(yu_ref, g_ref, out_ref):
    y3 = yu_ref[...].reshape(yu_ref.shape[0] // K, K, D)
    out_ref[...] = (g_ref[:, 0:1] * y3[:, 0, :] + g_ref[:, 1:2] * y3[:, 1, :])


def _combine(yu, g):
    TB = 256
    return pl.pallas_call(
        _combine_body,
        grid=(T // TB,),
        in_specs=[
            pl.BlockSpec((TB * K, D), lambda i: (i, 0)),
            pl.BlockSpec((TB, K), lambda i: (i, 0)),
        ],
        out_specs=pl.BlockSpec((TB, D), lambda i: (i, 0)),
        out_shape=jax.ShapeDtypeStruct((T, D), jnp.float32),
    )(yu, g)


def kernel(moe_inp, gate_w, gate_b, w1, b1, w2, b2):
    topi, g = _gate(moe_inp, gate_w, gate_b)
    row_gather, be, dest_slot, nb = _routing(topi)
    xs = jnp.take(moe_inp, row_gather, axis=0)
    ys = _ffn(xs, w1, b1, w2, b2, be, nb)
    yu = jnp.zeros((NPAD, D), jnp.float32).at[dest_slot].set(
        ys, mode="drop", unique_indices=False)
    return _combine(yu[:S], g)
